# baseline (speedup 1.0000x reference)
import jax
import jax.numpy as jnp
from jax.experimental import pallas as pl

B = 8192
K = 1024
D = 64
GRID = 8
STEP_B = B // GRID
CHUNK = 1024
N_CHUNKS = STEP_B // CHUNK
NSLAB = K // 8
STRIPES = 4
BIG = 3e38


def _vq_argmin_kernel(z_ref, w_ref, out_ref):
    w = w_ref[...]
    wm2 = -2.0 * w
    wsq = jnp.sum(w * w, axis=1, keepdims=True)
    siota = jax.lax.broadcasted_iota(
        jnp.int32, (8, CHUNK), 0).astype(jnp.float32)
    for i in range(N_CHUNKS):
        zc = z_ref[:, i * CHUNK:(i + 1) * CHUNK]
        cross2 = jax.lax.dot_general(
            wm2, zc, (((1,), (0,)), ((), ())),
            preferred_element_type=jnp.float32)
        ms = [jnp.full((8, CHUNK), BIG, jnp.float32)] * STRIPES
        bs = [jnp.zeros((8, CHUNK), jnp.float32)] * STRIPES
        for j in range(NSLAB):
            s = j % STRIPES
            slab = cross2[8 * j:8 * (j + 1), :] + wsq[8 * j:8 * (j + 1), :]
            take = slab < ms[s]
            bs[s] = jnp.where(take, jnp.float32(j), bs[s])
            ms[s] = jnp.minimum(ms[s], slab)
        m, bj = ms[0], bs[0]
        for s in range(1, STRIPES):
            pick = (ms[s] < m) | ((ms[s] == m) & (bs[s] < bj))
            bj = jnp.where(pick, bs[s], bj)
            m = jnp.minimum(m, ms[s])
        k8 = bj * 8.0 + siota
        mm = jnp.min(m, axis=0, keepdims=True)
        idx = jnp.min(jnp.where(m == mm, k8, jnp.float32(K)), axis=0)
        out_ref[:, i * CHUNK:(i + 1) * CHUNK] = idx.astype(jnp.int32)[None, :]


@jax.jit
def kernel(z_e_x, W):
    zT = z_e_x.T
    out = pl.pallas_call(
        _vq_argmin_kernel,
        grid=(GRID,),
        in_specs=[
            pl.BlockSpec((D, STEP_B), lambda g: (0, g)),
            pl.BlockSpec((K, D), lambda g: (0, 0)),
        ],
        out_specs=pl.BlockSpec((1, STEP_B), lambda g: (0, g)),
        out_shape=jax.ShapeDtypeStruct((1, B), jnp.int32),
    )(zT, W)
    return out.reshape(B)


# speedup vs baseline: 1.1442x; 1.1442x over previous
import jax
import jax.numpy as jnp
from jax.experimental import pallas as pl

B = 8192
K = 1024
D = 64
GRID = 2
STEP_B = B // GRID
CHUNK = 1024
N_CHUNKS = STEP_B // CHUNK
NSLAB = K // 8
STRIPES = 4
BIG = 3e38


def _vq_argmin_kernel(z_ref, w_ref, out_ref):
    w = w_ref[...]
    wm2 = -2.0 * w
    wsq = jnp.sum(w * w, axis=1, keepdims=True)
    siota = jax.lax.broadcasted_iota(
        jnp.int32, (8, CHUNK), 0).astype(jnp.float32)
    for i in range(N_CHUNKS):
        zc = z_ref[:, i * CHUNK:(i + 1) * CHUNK]
        cross2 = jax.lax.dot_general(
            wm2, zc, (((1,), (0,)), ((), ())),
            preferred_element_type=jnp.float32)
        ms = [jnp.full((8, CHUNK), BIG, jnp.float32)] * STRIPES
        bs = [jnp.zeros((8, CHUNK), jnp.float32)] * STRIPES
        for j in range(NSLAB):
            s = j % STRIPES
            slab = cross2[8 * j:8 * (j + 1), :] + wsq[8 * j:8 * (j + 1), :]
            take = slab < ms[s]
            bs[s] = jnp.where(take, jnp.float32(j), bs[s])
            ms[s] = jnp.minimum(ms[s], slab)
        m, bj = ms[0], bs[0]
        for s in range(1, STRIPES):
            pick = (ms[s] < m) | ((ms[s] == m) & (bs[s] < bj))
            bj = jnp.where(pick, bs[s], bj)
            m = jnp.minimum(m, ms[s])
        k8 = bj * 8.0 + siota
        mm = jnp.min(m, axis=0, keepdims=True)
        idx = jnp.min(jnp.where(m == mm, k8, jnp.float32(K)), axis=0)
        out_ref[:, i * CHUNK:(i + 1) * CHUNK] = idx.astype(jnp.int32)[None, :]


@jax.jit
def kernel(z_e_x, W):
    zT = z_e_x.T
    out = pl.pallas_call(
        _vq_argmin_kernel,
        grid=(GRID,),
        in_specs=[
            pl.BlockSpec((D, STEP_B), lambda g: (0, g)),
            pl.BlockSpec((K, D), lambda g: (0, 0)),
        ],
        out_specs=pl.BlockSpec((1, STEP_B), lambda g: (0, g)),
        out_shape=jax.ShapeDtypeStruct((1, B), jnp.int32),
    )(zT, W)
    return out.reshape(B)
